# baseline (speedup 1.0000x reference)
import jax
import jax.numpy as jnp
import numpy as np
from jax.experimental import pallas as pl

B, T, C, H, W = 16, 16, 96, 114, 10
HW = H * W
HWP = 1152
BT = B * T
K_SP = max(1, int((1.0 - 0.333) * HW))
K_CH = max(1, int((1.0 - 0.333) * C))
NUM_APPLY = max(1, int(B * 0.333))

try:
    _perm = np.asarray(jax.random.permutation(jax.random.key(42), B))
except Exception:
    _perm = np.array([7, 4, 2, 5, 3, 6, 10, 11, 15, 8, 9, 13, 14, 0, 1, 12])
SEL = tuple(int(b) for b in _perm[:NUM_APPLY])
NSEL = len(SEL)
P = NSEL * T
TH = T // 2

_F32_INF_BITS = 0x7F800000


def _selb(i):
    b = jnp.int32(SEL[0])
    for j in range(1, NSEL):
        b = jnp.where(i == j, jnp.int32(SEL[j]), b)
    return b


def _reduce_body(g_ref, sp_ref, ch_ref):
    a = jnp.abs(g_ref[0])
    sp = jnp.sum(a, axis=1)
    pad = jnp.full((TH, HWP - HW), jnp.inf, jnp.float32)
    sp_ref[...] = jnp.concatenate([sp, pad], axis=1)
    ch_ref[...] = jnp.sum(a, axis=2)


def _kth_bits(x_bits, k):
    rows = x_bits.shape[0]
    lo = jnp.zeros((rows, 1), jnp.int32)
    hi = jnp.full((rows, 1), _F32_INF_BITS, jnp.int32)

    def body(_, carry):
        lo, hi = carry
        mid = lo + (hi - lo) // 2
        cnt = jnp.sum((x_bits <= mid).astype(jnp.int32), axis=1,
                      keepdims=True)
        ge = cnt >= k
        return jnp.where(ge, lo, mid + 1), jnp.where(ge, mid, hi)

    lo, hi = jax.lax.fori_loop(0, 31, body, (lo, hi))
    return hi


def _thresh_body(sp_ref, ch_ref, spm_ref, chm_ref):
    x = jax.lax.bitcast_convert_type(sp_ref[...], jnp.int32)
    spm = (x < _kth_bits(x, K_SP)).astype(jnp.float32)

    y = jax.lax.bitcast_convert_type(ch_ref[...], jnp.int32)
    chm = (y < _kth_bits(y, K_CH)).astype(jnp.float32)

    spm_ref[...] = jnp.ones((BT, HWP), jnp.float32)
    chm_ref[...] = jnp.ones((BT, C), jnp.float32)
    for i, b in enumerate(SEL):
        spm_ref[b * T:(b + 1) * T, :] = spm[i * T:(i + 1) * T, :]
        chm_ref[b * T:(b + 1) * T, :] = chm[i * T:(i + 1) * T, :]


def _apply_body(z_ref, spm_ref, chm_ref, out_ref):
    sp_row = spm_ref[0][:, :HW]
    ch_row = chm_ref[0]
    ch_col = jax.lax.dot_general(
        ch_row, jnp.ones((1, 1), jnp.float32),
        (((0,), (0,)), ((), ())),
        preferred_element_type=jnp.float32)
    out_ref[0] = z_ref[0] * sp_row * ch_col


def kernel(z_local, gradient):
    g = gradient.reshape(B, T, C, HW)
    z = z_local.reshape(BT, C, HW)
    f32 = jnp.float32

    sp_sel, ch_sel = pl.pallas_call(
        _reduce_body,
        grid=(NSEL, T // TH),
        in_specs=[pl.BlockSpec((1, TH, C, HW),
                               lambda i, h: (_selb(i), h, 0, 0))],
        out_specs=[
            pl.BlockSpec((TH, HWP), lambda i, h: (i * (T // TH) + h, 0)),
            pl.BlockSpec((TH, C), lambda i, h: (i * (T // TH) + h, 0)),
        ],
        out_shape=[
            jax.ShapeDtypeStruct((P, HWP), f32),
            jax.ShapeDtypeStruct((P, C), f32),
        ],
    )(g)

    spm, chm = pl.pallas_call(
        _thresh_body,
        out_shape=[
            jax.ShapeDtypeStruct((BT, HWP), f32),
            jax.ShapeDtypeStruct((BT, C), f32),
        ],
    )(sp_sel, ch_sel)

    out = pl.pallas_call(
        _apply_body,
        grid=(BT,),
        in_specs=[
            pl.BlockSpec((1, C, HW), lambda p: (p, 0, 0)),
            pl.BlockSpec((1, 1, HWP), lambda p: (p, 0, 0)),
            pl.BlockSpec((1, 1, C), lambda p: (p, 0, 0)),
        ],
        out_specs=pl.BlockSpec((1, C, HW), lambda p: (p, 0, 0)),
        out_shape=jax.ShapeDtypeStruct((BT, C, HW), f32),
    )(z, spm.reshape(BT, 1, HWP), chm.reshape(BT, 1, C))

    return out.reshape(B, T, C, H, W)


# speedup vs baseline: 4.4487x; 4.4487x over previous
import jax
import jax.numpy as jnp
import numpy as np
from jax.experimental import pallas as pl

B, T, C, H, W = 16, 16, 96, 114, 10
HW = H * W
BT = B * T
K_SP = max(1, int((1.0 - 0.333) * HW))
K_CH = max(1, int((1.0 - 0.333) * C))
NUM_APPLY = max(1, int(B * 0.333))

try:
    _perm = np.asarray(jax.random.permutation(jax.random.key(42), B))
except Exception:
    _perm = np.array([7, 4, 2, 5, 3, 6, 10, 11, 15, 8, 9, 13, 14, 0, 1, 12])
SEL = tuple(int(b) for b in _perm[:NUM_APPLY])
NSEL = len(SEL)
P = NSEL * T
TH = T // 2

_F32_INF_BITS = 0x7F800000


def _selb(i):
    b = jnp.int32(SEL[0])
    for j in range(1, NSEL):
        b = jnp.where(i == j, jnp.int32(SEL[j]), b)
    return b


def _reduce_body(g_ref, sp_ref, ch_ref):
    a = jnp.abs(g_ref[0])
    sp_ref[...] = jnp.sum(a, axis=2)
    ch_ref[...] = jnp.sum(jnp.sum(a, axis=1), axis=2)


def _kth_bits(x_bits, k, axes, red_shape):
    lo = jnp.zeros(red_shape, jnp.int32)
    hi = jnp.full(red_shape, _F32_INF_BITS, jnp.int32)

    def body(_, carry):
        lo, hi = carry
        mid = lo + (hi - lo) // 2
        cnt = (x_bits <= mid).astype(jnp.int32)
        for ax in axes:
            cnt = jnp.sum(cnt, axis=ax, keepdims=True)
        ge = cnt >= k
        return jnp.where(ge, lo, mid + 1), jnp.where(ge, mid, hi)

    lo, hi = jax.lax.fori_loop(0, 31, body, (lo, hi))
    return hi


def _thresh_body(sp_ref, ch_ref, spm_ref, chm_ref):
    x = jax.lax.bitcast_convert_type(sp_ref[...], jnp.int32)
    thr_sp = _kth_bits(x, K_SP, (2, 1), (P, 1, 1))
    spm = (x < thr_sp).astype(jnp.float32)

    y = jax.lax.bitcast_convert_type(ch_ref[...], jnp.int32)
    thr_ch = _kth_bits(y, K_CH, (1,), (P, 1))
    chm = (y < thr_ch).astype(jnp.float32)

    spm_ref[...] = jnp.ones((BT, W, H), jnp.float32)
    chm_ref[...] = jnp.ones((BT, C), jnp.float32)
    for i, b in enumerate(SEL):
        spm_ref[b * T:(b + 1) * T] = spm[i * T:(i + 1) * T]
        chm_ref[b * T:(b + 1) * T] = chm[i * T:(i + 1) * T]


def _apply_body(z_ref, spm_ref, chm_ref, out_ref):
    ch_row = chm_ref[0]
    ch_col = jax.lax.dot_general(
        ch_row, jnp.ones((1, 1), jnp.float32),
        (((0,), (0,)), ((), ())),
        preferred_element_type=jnp.float32)
    spv = spm_ref[0]
    for w in range(W):
        out_ref[0, w] = z_ref[0, w] * (ch_col * spv[w:w + 1, :])


def kernel(z_local, gradient):
    f32 = jnp.float32
    gt = gradient.transpose(0, 1, 4, 2, 3)
    zt = z_local.transpose(0, 1, 4, 2, 3).reshape(BT, W, C, H)

    sp_sel, ch_sel = pl.pallas_call(
        _reduce_body,
        grid=(NSEL, T // TH),
        in_specs=[pl.BlockSpec((1, TH, W, C, H),
                               lambda i, h: (_selb(i), h, 0, 0, 0))],
        out_specs=[
            pl.BlockSpec((TH, W, H), lambda i, h: (i * (T // TH) + h, 0, 0)),
            pl.BlockSpec((TH, C), lambda i, h: (i * (T // TH) + h, 0)),
        ],
        out_shape=[
            jax.ShapeDtypeStruct((P, W, H), f32),
            jax.ShapeDtypeStruct((P, C), f32),
        ],
    )(gt)

    spm, chm = pl.pallas_call(
        _thresh_body,
        out_shape=[
            jax.ShapeDtypeStruct((BT, W, H), f32),
            jax.ShapeDtypeStruct((BT, C), f32),
        ],
    )(sp_sel, ch_sel)

    out = pl.pallas_call(
        _apply_body,
        grid=(BT,),
        in_specs=[
            pl.BlockSpec((1, W, C, H), lambda p: (p, 0, 0, 0)),
            pl.BlockSpec((1, W, H), lambda p: (p, 0, 0)),
            pl.BlockSpec((1, 1, C), lambda p: (p, 0, 0)),
        ],
        out_specs=pl.BlockSpec((1, W, C, H), lambda p: (p, 0, 0, 0)),
        out_shape=jax.ShapeDtypeStruct((BT, W, C, H), f32),
    )(zt, spm, chm.reshape(BT, 1, C))

    return out.reshape(B, T, W, C, H).transpose(0, 1, 3, 4, 2)


# speedup vs baseline: 8.1356x; 1.8288x over previous
import jax
import jax.numpy as jnp
import numpy as np
from jax.experimental import pallas as pl

B, T, C, H, W = 16, 16, 96, 114, 10
HW = H * W
BT = B * T
K_SP = max(1, int((1.0 - 0.333) * HW))
K_CH = max(1, int((1.0 - 0.333) * C))
NUM_APPLY = max(1, int(B * 0.333))

try:
    _perm = np.asarray(jax.random.permutation(jax.random.key(42), B))
except Exception:
    _perm = np.array([7, 4, 2, 5, 3, 6, 10, 11, 15, 8, 9, 13, 14, 0, 1, 12])
SEL = tuple(int(b) for b in _perm[:NUM_APPLY])
NONSEL = tuple(b for b in range(B) if b not in SEL)
NSEL = len(SEL)
P = NSEL * T
TH = T // 2
NH = T // TH
RB = 4

_F32_INF_BITS = 0x7F800000


def _selb(i):
    b = jnp.int32(SEL[0])
    for j in range(1, NSEL):
        b = jnp.where(i == j, jnp.int32(SEL[j]), b)
    return b


def _kth_bits(x_bits, k, axes, red_shape):
    lo = jnp.zeros(red_shape, jnp.int32)
    hi = jnp.full(red_shape, _F32_INF_BITS, jnp.int32)

    def body(_, carry):
        lo, hi = carry
        mid = lo + (hi - lo) // 2
        cnt = (x_bits <= mid).astype(jnp.int32)
        for ax in axes:
            cnt = jnp.sum(cnt, axis=ax, keepdims=True)
        ge = cnt >= k
        return jnp.where(ge, lo, mid + 1), jnp.where(ge, mid, hi)

    lo, hi = jax.lax.fori_loop(0, 31, body, (lo, hi))
    return hi


def _mask_body(g_ref, spm_ref, chm_ref):
    i = pl.program_id(0)
    h = pl.program_id(1)
    a = jnp.abs(g_ref[0])
    base = _selb(i) * T + h * TH
    spm_ref[pl.ds(base, TH)] = jnp.sum(a, axis=2)
    chm_ref[pl.ds(base, TH)] = jnp.sum(jnp.sum(a, axis=1), axis=2)

    @pl.when(jnp.logical_and(i == NSEL - 1, h == NH - 1))
    def _finalize():
        x = jnp.concatenate(
            [jax.lax.bitcast_convert_type(spm_ref[b * T:(b + 1) * T],
                                          jnp.int32) for b in SEL], axis=0)
        thr_sp = _kth_bits(x, K_SP, (2, 1), (P, 1, 1))
        spm = (x < thr_sp).astype(jnp.float32)

        y = jnp.concatenate(
            [jax.lax.bitcast_convert_type(chm_ref[b * T:(b + 1) * T],
                                          jnp.int32) for b in SEL], axis=0)
        thr_ch = _kth_bits(y, K_CH, (1,), (P, 1))
        chm = (y < thr_ch).astype(jnp.float32)

        for j, b in enumerate(SEL):
            spm_ref[b * T:(b + 1) * T] = spm[j * T:(j + 1) * T]
            chm_ref[b * T:(b + 1) * T] = chm[j * T:(j + 1) * T]
        for b in NONSEL:
            spm_ref[b * T:(b + 1) * T] = jnp.ones((T, W, H), jnp.float32)
            chm_ref[b * T:(b + 1) * T] = jnp.ones((T, C), jnp.float32)


def _apply_body(z_ref, spm_ref, chm_ref, out_ref):
    spv = spm_ref[...]
    for r in range(RB):
        ch_col = jax.lax.dot_general(
            chm_ref[r], jnp.ones((1, 1), jnp.float32),
            (((0,), (0,)), ((), ())),
            preferred_element_type=jnp.float32)
        for w in range(W):
            out_ref[r, w] = z_ref[r, w] * (ch_col * spv[r, w:w + 1, :])


def kernel(z_local, gradient):
    f32 = jnp.float32
    gt = gradient.transpose(0, 1, 4, 2, 3)
    zt = z_local.transpose(0, 1, 4, 2, 3).reshape(BT, W, C, H)

    spm, chm = pl.pallas_call(
        _mask_body,
        grid=(NSEL, NH),
        in_specs=[pl.BlockSpec((1, TH, W, C, H),
                               lambda i, h: (_selb(i), h, 0, 0, 0))],
        out_specs=[
            pl.BlockSpec((BT, W, H), lambda i, h: (0, 0, 0)),
            pl.BlockSpec((BT, C), lambda i, h: (0, 0)),
        ],
        out_shape=[
            jax.ShapeDtypeStruct((BT, W, H), f32),
            jax.ShapeDtypeStruct((BT, C), f32),
        ],
    )(gt)

    out = pl.pallas_call(
        _apply_body,
        grid=(BT // RB,),
        in_specs=[
            pl.BlockSpec((RB, W, C, H), lambda p: (p, 0, 0, 0)),
            pl.BlockSpec((RB, W, H), lambda p: (p, 0, 0)),
            pl.BlockSpec((RB, 1, C), lambda p: (p, 0, 0)),
        ],
        out_specs=pl.BlockSpec((RB, W, C, H), lambda p: (p, 0, 0, 0)),
        out_shape=jax.ShapeDtypeStruct((BT, W, C, H), f32),
    )(zt, spm, chm.reshape(BT, 1, C))

    return out.reshape(B, T, W, C, H).transpose(0, 1, 3, 4, 2)


# speedup vs baseline: 8.2545x; 1.0146x over previous
import jax
import jax.numpy as jnp
import numpy as np
from jax.experimental import pallas as pl
from jax.experimental.pallas import tpu as pltpu

B, T, C, H, W = 16, 16, 96, 114, 10
HW = H * W
BT = B * T
K_SP = max(1, int((1.0 - 0.333) * HW))
K_CH = max(1, int((1.0 - 0.333) * C))
NUM_APPLY = max(1, int(B * 0.333))

try:
    _perm = np.asarray(jax.random.permutation(jax.random.key(42), B))
except Exception:
    _perm = np.array([7, 4, 2, 5, 3, 6, 10, 11, 15, 8, 9, 13, 14, 0, 1, 12])
SEL = tuple(int(b) for b in _perm[:NUM_APPLY])
NONSEL = tuple(b for b in range(B) if b not in SEL)
NSEL = len(SEL)
P = NSEL * T
TH = T // 2
NH = T // TH
NPRE = NSEL * NH
RB = 4

_F32_INF_BITS = 0x7F800000


def _selb(i):
    b = jnp.int32(SEL[0])
    for j in range(1, NSEL):
        b = jnp.where(i == j, jnp.int32(SEL[j]), b)
    return b


def _kth_bits(x_bits, k, axes, red_shape):
    lo = jnp.zeros(red_shape, jnp.int32)
    hi = jnp.full(red_shape, _F32_INF_BITS, jnp.int32)

    def body(_, carry):
        lo, hi = carry
        mid = lo + (hi - lo) // 2
        cnt = (x_bits <= mid).astype(jnp.int32)
        for ax in axes:
            cnt = jnp.sum(cnt, axis=ax, keepdims=True)
        ge = cnt >= k
        return jnp.where(ge, lo, mid + 1), jnp.where(ge, mid, hi)

    lo, hi = jax.lax.fori_loop(0, 31, body, (lo, hi))
    return hi


def _fused_body(g_ref, z_ref, out_ref, spm_s, chm_s):
    p = pl.program_id(0)

    @pl.when(p < NPRE)
    def _reduce():
        i = p // NH
        h = p % NH
        a = jnp.abs(g_ref[0])
        base = _selb(i) * T + h * TH
        spm_s[pl.ds(base, TH)] = jnp.sum(a, axis=2)
        chm_s[pl.ds(base, TH)] = jnp.sum(jnp.sum(a, axis=1), axis=2)

    @pl.when(p == NPRE - 1)
    def _finalize():
        x = jnp.concatenate(
            [jax.lax.bitcast_convert_type(spm_s[b * T:(b + 1) * T],
                                          jnp.int32) for b in SEL], axis=0)
        thr_sp = _kth_bits(x, K_SP, (2, 1), (P, 1, 1))
        spm = (x < thr_sp).astype(jnp.float32)

        y = jnp.concatenate(
            [jax.lax.bitcast_convert_type(chm_s[b * T:(b + 1) * T],
                                          jnp.int32) for b in SEL], axis=0)
        thr_ch = _kth_bits(y, K_CH, (1,), (P, 1))
        chm = (y < thr_ch).astype(jnp.float32)

        for j, b in enumerate(SEL):
            spm_s[b * T:(b + 1) * T] = spm[j * T:(j + 1) * T]
            chm_s[b * T:(b + 1) * T] = chm[j * T:(j + 1) * T]
        for b in NONSEL:
            spm_s[b * T:(b + 1) * T] = jnp.ones((T, W, H), jnp.float32)
            chm_s[b * T:(b + 1) * T] = jnp.ones((T, C), jnp.float32)

    @pl.when(p >= NPRE)
    def _apply():
        base = (p - NPRE) * RB
        spv = spm_s[pl.ds(base, RB)]
        chv = chm_s[pl.ds(base, RB)]
        for r in range(RB):
            ch_col = jax.lax.dot_general(
                chv[r:r + 1], jnp.ones((1, 1), jnp.float32),
                (((0,), (0,)), ((), ())),
                preferred_element_type=jnp.float32)
            for w in range(W):
                out_ref[r, w] = z_ref[r, w] * (ch_col * spv[r, w:w + 1, :])


def _g_im(p):
    pp = jnp.minimum(p, NPRE - 1)
    return (_selb(pp // NH), pp % NH, 0, 0, 0)


def _z_im(p):
    return (jnp.maximum(p - NPRE, 0), 0, 0, 0)


def kernel(z_local, gradient):
    f32 = jnp.float32
    gt = gradient.transpose(0, 1, 4, 2, 3)
    zt = z_local.transpose(0, 1, 4, 2, 3).reshape(BT, W, C, H)

    out = pl.pallas_call(
        _fused_body,
        grid=(NPRE + BT // RB,),
        in_specs=[
            pl.BlockSpec((1, TH, W, C, H), _g_im),
            pl.BlockSpec((RB, W, C, H), _z_im),
        ],
        out_specs=pl.BlockSpec((RB, W, C, H), _z_im),
        out_shape=jax.ShapeDtypeStruct((BT, W, C, H), f32),
        scratch_shapes=[
            pltpu.VMEM((BT, W, H), f32),
            pltpu.VMEM((BT, C), f32),
        ],
    )(gt, zt)

    return out.reshape(B, T, W, C, H).transpose(0, 1, 3, 4, 2)
